# baseline (speedup 1.0000x reference)
import jax
import jax.numpy as jnp
from jax import lax
from jax.experimental import pallas as pl
from jax.experimental.pallas import tpu as pltpu
from jax.experimental.pallas import tpu_sc as plsc

NUM_BILLS = 1000000
NUM_LEGS = 100000
BATCH = 16384
LATENT_DIM = 16
NUM_WORKERS = 32
PAIRS_PER_WORKER = BATCH // NUM_WORKERS
CHUNK = 128
CHUNKS_PER_WORKER = PAIRS_PER_WORKER // CHUNK
GROUPS = PAIRS_PER_WORKER // 16


def _sc_body(bids, lids, gb, leg_b, bill_b, leg_t, bill_t, out_hbm,
             bidx, lidx, bgidx, lgidx, bcols, lcols, bb_v, lb_v, gb_v,
             out_v, sem):
    wid = lax.axis_index("s") * 2 + lax.axis_index("c")
    base = wid * PAIRS_PER_WORKER

    pltpu.sync_copy(bids.at[pl.ds(base, PAIRS_PER_WORKER)], bidx)
    pltpu.sync_copy(lids.at[pl.ds(base, PAIRS_PER_WORKER)], lidx)
    pltpu.sync_copy(gb, gb_v)

    bias_copies = []
    for c in range(CHUNKS_PER_WORKER):
        sl = pl.ds(c * CHUNK, CHUNK)
        bias_copies.append(pltpu.async_copy(
            bill_b.at[bidx.at[sl]], bb_v.at[sl], sem))
        bias_copies.append(pltpu.async_copy(
            leg_b.at[lidx.at[sl]], lb_v.at[sl], sem))

    def build(v, _):
        sl = pl.ds(v * 16, 16)
        bv = bidx[sl]
        lv = lidx[sl]
        for d in range(LATENT_DIM):
            bgidx[d, sl] = bv + d * NUM_BILLS
            lgidx[d, sl] = lv + d * NUM_LEGS
        return 0

    lax.fori_loop(0, GROUPS, build, 0, unroll=False)

    for cp in bias_copies:
        cp.wait()

    def gather_dim(d, _):
        copies = []
        for c in range(CHUNKS_PER_WORKER):
            sl = pl.ds(c * CHUNK, CHUNK)
            copies.append(pltpu.async_copy(
                bill_t.at[bgidx.at[d, sl]], bcols.at[d, sl], sem))
            copies.append(pltpu.async_copy(
                leg_t.at[lgidx.at[d, sl]], lcols.at[d, sl], sem))
        for cp in copies:
            cp.wait()
        return 0

    lax.fori_loop(0, LATENT_DIM, gather_dim, 0, unroll=False)

    gbv = gb_v[...]

    def group(g, _):
        sl = pl.ds(g * 16, 16)
        acc = bcols[0, sl] * lcols[0, sl]
        for d in range(1, LATENT_DIM):
            acc = acc + bcols[d, sl] * lcols[d, sl]
        x = gbv + bb_v[sl] + lb_v[sl] + acc
        out_v[sl] = 1.0 / (1.0 + jnp.exp(-x))
        return 0

    lax.fori_loop(0, GROUPS, group, 0, unroll=False)

    pltpu.sync_copy(out_v, out_hbm.at[pl.ds(base, PAIRS_PER_WORKER)])


@jax.jit
def _predict(bids, lids, gb, leg_b, bill_b, leg_t, bill_t):
    mesh = plsc.VectorSubcoreMesh(core_axis_name="c", subcore_axis_name="s")
    k = pl.kernel(
        _sc_body,
        out_type=jax.ShapeDtypeStruct((BATCH,), jnp.float32),
        mesh=mesh,
        compiler_params=pltpu.CompilerParams(needs_layout_passes=False,
                                             use_tc_tiling_on_sc=False),
        scratch_types=[
            pltpu.VMEM((PAIRS_PER_WORKER,), jnp.int32),
            pltpu.VMEM((PAIRS_PER_WORKER,), jnp.int32),
            pltpu.VMEM((LATENT_DIM, PAIRS_PER_WORKER), jnp.int32),
            pltpu.VMEM((LATENT_DIM, PAIRS_PER_WORKER), jnp.int32),
            pltpu.VMEM((LATENT_DIM, PAIRS_PER_WORKER), jnp.float32),
            pltpu.VMEM((LATENT_DIM, PAIRS_PER_WORKER), jnp.float32),
            pltpu.VMEM((PAIRS_PER_WORKER,), jnp.float32),
            pltpu.VMEM((PAIRS_PER_WORKER,), jnp.float32),
            pltpu.VMEM((16,), jnp.float32),
            pltpu.VMEM((PAIRS_PER_WORKER,), jnp.float32),
            pltpu.SemaphoreType.DMA,
        ],
    )
    return k(bids, lids, gb, leg_b, bill_b, leg_t, bill_t)


def kernel(bill_ids, legislator_ids, global_bias, legislator_bias, bill_bias,
           legislator_embedding, bill_embedding):
    bids = bill_ids.astype(jnp.int32)
    lids = legislator_ids.astype(jnp.int32)
    gb = jnp.broadcast_to(jnp.reshape(global_bias, (1,)), (16,))
    leg_b = jnp.reshape(legislator_bias, (-1,))
    bill_b = jnp.reshape(bill_bias, (-1,))
    leg_t = jnp.ravel(jnp.transpose(legislator_embedding))
    bill_t = jnp.ravel(jnp.transpose(bill_embedding))
    return _predict(bids, lids, gb, leg_b, bill_b, leg_t, bill_t)


# speedup vs baseline: 1.0032x; 1.0032x over previous
import jax
import jax.numpy as jnp
from jax import lax
from jax.experimental import pallas as pl
from jax.experimental.pallas import tpu as pltpu
from jax.experimental.pallas import tpu_sc as plsc

BATCH = 16384
LATENT_DIM = 16
NUM_WORKERS = 32
PAIRS_PER_WORKER = BATCH // NUM_WORKERS
CHUNK = 128
CHUNKS_PER_WORKER = PAIRS_PER_WORKER // CHUNK
GROUPS = PAIRS_PER_WORKER // 16


def _sc_body(bids, lids, gb, leg_b, bill_b, leg_t, bill_t, out_hbm,
             bidx, lidx, bcols, lcols, bb_v, lb_v, gb_v, out_v, sem):
    wid = lax.axis_index("s") * 2 + lax.axis_index("c")
    base = wid * PAIRS_PER_WORKER

    pltpu.sync_copy(bids.at[pl.ds(base, PAIRS_PER_WORKER)], bidx)
    pltpu.sync_copy(lids.at[pl.ds(base, PAIRS_PER_WORKER)], lidx)
    pltpu.sync_copy(gb, gb_v)

    bias_copies = []
    for c in range(CHUNKS_PER_WORKER):
        sl = pl.ds(c * CHUNK, CHUNK)
        bias_copies.append(pltpu.async_copy(
            bill_b.at[bidx.at[sl]], bb_v.at[sl], sem))
        bias_copies.append(pltpu.async_copy(
            leg_b.at[lidx.at[sl]], lb_v.at[sl], sem))

    def gather_dim(d, _):
        copies = []
        for c in range(CHUNKS_PER_WORKER):
            sl = pl.ds(c * CHUNK, CHUNK)
            copies.append(pltpu.async_copy(
                bill_t.at[d].at[bidx.at[sl]], bcols.at[d, sl], sem))
            copies.append(pltpu.async_copy(
                leg_t.at[d].at[lidx.at[sl]], lcols.at[d, sl], sem))
        for cp in copies:
            cp.wait()
        return 0

    lax.fori_loop(0, LATENT_DIM, gather_dim, 0, unroll=False)

    for cp in bias_copies:
        cp.wait()

    gbv = gb_v[...]

    def group(g, _):
        sl = pl.ds(g * 16, 16)
        acc = bcols[0, sl] * lcols[0, sl]
        for d in range(1, LATENT_DIM):
            acc = acc + bcols[d, sl] * lcols[d, sl]
        x = gbv + bb_v[sl] + lb_v[sl] + acc
        out_v[sl] = 1.0 / (1.0 + jnp.exp(-x))
        return 0

    lax.fori_loop(0, GROUPS, group, 0, unroll=False)

    pltpu.sync_copy(out_v, out_hbm.at[pl.ds(base, PAIRS_PER_WORKER)])


@jax.jit
def _predict(bids, lids, gb, leg_b, bill_b, leg_t, bill_t):
    mesh = plsc.VectorSubcoreMesh(core_axis_name="c", subcore_axis_name="s")
    k = pl.kernel(
        _sc_body,
        out_type=jax.ShapeDtypeStruct((BATCH,), jnp.float32),
        mesh=mesh,
        compiler_params=pltpu.CompilerParams(needs_layout_passes=False,
                                             use_tc_tiling_on_sc=False),
        scratch_types=[
            pltpu.VMEM((PAIRS_PER_WORKER,), jnp.int32),
            pltpu.VMEM((PAIRS_PER_WORKER,), jnp.int32),
            pltpu.VMEM((LATENT_DIM, PAIRS_PER_WORKER), jnp.float32),
            pltpu.VMEM((LATENT_DIM, PAIRS_PER_WORKER), jnp.float32),
            pltpu.VMEM((PAIRS_PER_WORKER,), jnp.float32),
            pltpu.VMEM((PAIRS_PER_WORKER,), jnp.float32),
            pltpu.VMEM((16,), jnp.float32),
            pltpu.VMEM((PAIRS_PER_WORKER,), jnp.float32),
            pltpu.SemaphoreType.DMA,
        ],
    )
    return k(bids, lids, gb, leg_b, bill_b, leg_t, bill_t)


def kernel(bill_ids, legislator_ids, global_bias, legislator_bias, bill_bias,
           legislator_embedding, bill_embedding):
    bids = bill_ids.astype(jnp.int32)
    lids = legislator_ids.astype(jnp.int32)
    gb = jnp.broadcast_to(jnp.reshape(global_bias, (1,)), (16,))
    leg_b = jnp.reshape(legislator_bias, (-1,))
    bill_b = jnp.reshape(bill_bias, (-1,))
    leg_t = jnp.transpose(legislator_embedding)
    bill_t = jnp.transpose(bill_embedding)
    return _predict(bids, lids, gb, leg_b, bill_b, leg_t, bill_t)


# speedup vs baseline: 6.8621x; 6.8403x over previous
import functools

import jax
import jax.numpy as jnp
from jax import lax
from jax.experimental import pallas as pl
from jax.experimental.pallas import tpu as pltpu
from jax.experimental.pallas import tpu_sc as plsc

NUM_BILLS = 1000000
NUM_LEGS = 100000
BILL_STRIDE = 1007616
LEG_STRIDE = 102400
BILL_BLK = BILL_STRIDE // 8
LEG_BLK = LEG_STRIDE // 2

BATCH = 16384
LATENT_DIM = 16
NUM_WORKERS = 32
PAIRS_PER_WORKER = BATCH // NUM_WORKERS
CHUNK = 128
CHUNKS_PER_WORKER = PAIRS_PER_WORKER // CHUNK
GROUPS = PAIRS_PER_WORKER // 16


def _flatten_body(t_ref, out_ref):
    out_ref[...] = t_ref[pl.program_id(1), :]


def _dim_major_flat(table_t, blk, blocks_per_row, stride):
    return pl.pallas_call(
        _flatten_body,
        grid=(blocks_per_row, LATENT_DIM),
        in_specs=[pl.BlockSpec((LATENT_DIM, blk), lambda j, d: (0, j))],
        out_specs=pl.BlockSpec(
            (blk,), lambda j, d: (d * blocks_per_row + j,)),
        out_shape=jax.ShapeDtypeStruct((LATENT_DIM * stride,), jnp.float32),
    )(table_t)


def _sc_body(bids, lids, gb, leg_b, bill_b, leg_t, bill_t, out_hbm,
             bidx, lidx, bgidx, lgidx, bcols, lcols, bb_v, lb_v, gb_v,
             out_v, sem):
    wid = lax.axis_index("s") * 2 + lax.axis_index("c")
    base = wid * PAIRS_PER_WORKER

    pltpu.sync_copy(bids.at[pl.ds(base, PAIRS_PER_WORKER)], bidx)
    pltpu.sync_copy(lids.at[pl.ds(base, PAIRS_PER_WORKER)], lidx)
    pltpu.sync_copy(gb, gb_v)

    bias_copies = []
    for c in range(CHUNKS_PER_WORKER):
        sl = pl.ds(c * CHUNK, CHUNK)
        bias_copies.append(pltpu.async_copy(
            bill_b.at[bidx.at[sl]], bb_v.at[sl], sem))
        bias_copies.append(pltpu.async_copy(
            leg_b.at[lidx.at[sl]], lb_v.at[sl], sem))

    def build(v, _):
        sl = pl.ds(v * 16, 16)
        bv = bidx[sl]
        lv = lidx[sl]
        for d in range(LATENT_DIM):
            bgidx[d, sl] = bv + d * BILL_STRIDE
            lgidx[d, sl] = lv + d * LEG_STRIDE
        return 0

    lax.fori_loop(0, GROUPS, build, 0, unroll=False)

    for cp in bias_copies:
        cp.wait()

    def gather_dim(d, _):
        copies = []
        for c in range(CHUNKS_PER_WORKER):
            sl = pl.ds(c * CHUNK, CHUNK)
            copies.append(pltpu.async_copy(
                bill_t.at[bgidx.at[d, sl]], bcols.at[d, sl], sem))
            copies.append(pltpu.async_copy(
                leg_t.at[lgidx.at[d, sl]], lcols.at[d, sl], sem))
        for cp in copies:
            cp.wait()
        return 0

    lax.fori_loop(0, LATENT_DIM, gather_dim, 0, unroll=False)

    gbv = gb_v[...]

    def group(g, _):
        sl = pl.ds(g * 16, 16)
        acc = bcols[0, sl] * lcols[0, sl]
        for d in range(1, LATENT_DIM):
            acc = acc + bcols[d, sl] * lcols[d, sl]
        x = gbv + bb_v[sl] + lb_v[sl] + acc
        out_v[sl] = 1.0 / (1.0 + jnp.exp(-x))
        return 0

    lax.fori_loop(0, GROUPS, group, 0, unroll=False)

    pltpu.sync_copy(out_v, out_hbm.at[pl.ds(base, PAIRS_PER_WORKER)])


@jax.jit
def _predict(bids, lids, gb, leg_b, bill_b, leg_emb_t, bill_emb_t):
    leg_t = _dim_major_flat(leg_emb_t, LEG_BLK, 2, LEG_STRIDE)
    bill_t = _dim_major_flat(bill_emb_t, BILL_BLK, 8, BILL_STRIDE)

    mesh = plsc.VectorSubcoreMesh(core_axis_name="c", subcore_axis_name="s")
    k = pl.kernel(
        _sc_body,
        out_type=jax.ShapeDtypeStruct((BATCH,), jnp.float32),
        mesh=mesh,
        compiler_params=pltpu.CompilerParams(needs_layout_passes=False,
                                             use_tc_tiling_on_sc=False),
        scratch_types=[
            pltpu.VMEM((PAIRS_PER_WORKER,), jnp.int32),
            pltpu.VMEM((PAIRS_PER_WORKER,), jnp.int32),
            pltpu.VMEM((LATENT_DIM, PAIRS_PER_WORKER), jnp.int32),
            pltpu.VMEM((LATENT_DIM, PAIRS_PER_WORKER), jnp.int32),
            pltpu.VMEM((LATENT_DIM, PAIRS_PER_WORKER), jnp.float32),
            pltpu.VMEM((LATENT_DIM, PAIRS_PER_WORKER), jnp.float32),
            pltpu.VMEM((PAIRS_PER_WORKER,), jnp.float32),
            pltpu.VMEM((PAIRS_PER_WORKER,), jnp.float32),
            pltpu.VMEM((16,), jnp.float32),
            pltpu.VMEM((PAIRS_PER_WORKER,), jnp.float32),
            pltpu.SemaphoreType.DMA,
        ],
    )
    return k(bids, lids, gb, leg_b, bill_b, leg_t, bill_t)


def kernel(bill_ids, legislator_ids, global_bias, legislator_bias, bill_bias,
           legislator_embedding, bill_embedding):
    bids = bill_ids.astype(jnp.int32)
    lids = legislator_ids.astype(jnp.int32)
    gb = jnp.broadcast_to(jnp.reshape(global_bias, (1,)), (16,))
    leg_b = jnp.reshape(legislator_bias, (-1,))
    bill_b = jnp.reshape(bill_bias, (-1,))
    leg_emb_t = jnp.transpose(legislator_embedding)
    bill_emb_t = jnp.transpose(bill_embedding)
    return _predict(bids, lids, gb, leg_b, bill_b, leg_emb_t, bill_emb_t)
